# baseline (speedup 1.0000x reference)
import jax
import jax.numpy as jnp
from jax.experimental import pallas as pl
from jax.experimental.pallas import tpu as pltpu


def _pe_kernel(a_ref, w_ref, pos_ref, cls_ref, b_ref, out_ref):
    TB = out_ref.shape[0]
    row0 = cls_ref[...] + pos_ref[0:1, :]
    pb = pos_ref[1:, :] + b_ref[...]
    for t in range(TB):
        emb = jnp.dot(a_ref[t].astype(jnp.bfloat16), w_ref[...],
                      preferred_element_type=jnp.float32)
        out_ref[t, 0:1, :] = row0
        out_ref[t, 1:, :] = emb + pb


def _vit_patch_embed(x, conv_w, conv_b, cls_token, pos_embed, patch_size,
                     *, batch_tile=8):
    B, C, H, W = x.shape
    ph, pw = patch_size
    gh, gw = H // ph, W // pw
    N = gh * gw
    D = conv_w.shape[0]
    K = C * ph * pw
    assert pos_embed.shape[1] == N + 1

    patches = x.reshape(B, C, gh, ph, gw, pw).transpose(0, 2, 4, 1, 3, 5)
    patches = patches.reshape(B, N, K)

    w_mat = conv_w.reshape(D, K).T.astype(jnp.bfloat16)

    TB = batch_tile
    grid = (B // TB,)

    out = pl.pallas_call(
        _pe_kernel,
        out_shape=jax.ShapeDtypeStruct((B, N + 1, D), x.dtype),
        grid_spec=pltpu.PrefetchScalarGridSpec(
            num_scalar_prefetch=0,
            grid=grid,
            in_specs=[
                pl.BlockSpec((TB, N, K), lambda b: (b, 0, 0)),
                pl.BlockSpec((K, D), lambda b: (0, 0)),
                pl.BlockSpec((N + 1, D), lambda b: (0, 0)),
                pl.BlockSpec((1, D), lambda b: (0, 0)),
                pl.BlockSpec((1, D), lambda b: (0, 0)),
            ],
            out_specs=pl.BlockSpec((TB, N + 1, D), lambda b: (b, 0, 0)),
        ),
        compiler_params=pltpu.CompilerParams(
            dimension_semantics=("parallel",),
            vmem_limit_bytes=100 * 1024 * 1024,
        ),
    )(patches, w_mat, pos_embed[0], cls_token.reshape(1, D),
      conv_b.reshape(1, D))
    return out


def kernel(x, conv_w, conv_b, cls_token, pos_embed):
    return _vit_patch_embed(x, conv_w, conv_b, cls_token, pos_embed, (16, 16))


# speedup vs baseline: 1.0511x; 1.0511x over previous
import jax
import jax.numpy as jnp
from jax.experimental import pallas as pl
from jax.experimental.pallas import tpu as pltpu


def _fused_patch_embed_kernel(x_ref, w_ref, pb_ref, out_ref):
    TB, C, H, W = x_ref.shape
    Mp, D = pb_ref.shape
    M = out_ref.shape[1]
    ph = pw = 16
    gh, gw = H // ph, W // pw
    N = gh * gw
    K = C * ph * pw

    xb = x_ref[...].astype(jnp.bfloat16)
    a = xb.reshape(TB, C, gh, ph, gw, pw)
    a = a.transpose(0, 2, 4, 1, 3, 5).reshape(TB, N, K)
    a = jnp.pad(a, ((0, 0), (1, Mp - 1 - N), (0, 0)))
    emb = jnp.dot(
        a.reshape(TB * Mp, K), w_ref[...],
        preferred_element_type=jnp.float32,
    ).reshape(TB, Mp, D)
    out_ref[...] = (emb[:, :M, :] + pb_ref[...][None, :M, :]).astype(out_ref.dtype)


def _vit_patch_embed(x, conv_w, conv_b, cls_token, pos_embed, patch_size,
                     *, batch_tile=8):
    B, C, H, W = x.shape
    ph, pw = patch_size
    gh, gw = H // ph, W // pw
    N = gh * gw
    D = conv_w.shape[0]
    K = C * ph * pw
    assert pos_embed.shape[1] == N + 1, "pos_embed / input size mismatch"

    Mp = ((N + 1 + 15) // 16) * 16

    w_mat = conv_w.reshape(D, K).T.astype(jnp.bfloat16)
    posbias = jnp.concatenate(
        [cls_token.reshape(1, D) + pos_embed[0, 0:1, :],
         pos_embed[0, 1:, :] + conv_b[None, :],
         jnp.zeros((Mp - 1 - N, D), jnp.float32)],
        axis=0,
    ).astype(jnp.float32)

    TB = batch_tile
    grid = (B // TB,)

    out = pl.pallas_call(
        _fused_patch_embed_kernel,
        out_shape=jax.ShapeDtypeStruct((B, N + 1, D), x.dtype),
        grid_spec=pltpu.PrefetchScalarGridSpec(
            num_scalar_prefetch=0,
            grid=grid,
            in_specs=[
                pl.BlockSpec((TB, C, H, W), lambda b: (b, 0, 0, 0)),
                pl.BlockSpec((K, D), lambda b: (0, 0)),
                pl.BlockSpec((Mp, D), lambda b: (0, 0)),
            ],
            out_specs=pl.BlockSpec((TB, N + 1, D), lambda b: (b, 0, 0)),
        ),
        compiler_params=pltpu.CompilerParams(
            dimension_semantics=("parallel",),
            vmem_limit_bytes=100 * 1024 * 1024,
        ),
    )(x, w_mat, posbias)
    return out


def kernel(x, conv_w, conv_b, cls_token, pos_embed):
    return _vit_patch_embed(x, conv_w, conv_b, cls_token, pos_embed, (16, 16))


# speedup vs baseline: 1.0680x; 1.0160x over previous
import jax
import jax.numpy as jnp
from jax.experimental import pallas as pl
from jax.experimental.pallas import tpu as pltpu


def _cast_kernel(x_ref, o_ref):
    o_ref[...] = x_ref[...].astype(jnp.bfloat16)


def _cast_bf16(x, tb):
    B, C, H, W = x.shape
    return pl.pallas_call(
        _cast_kernel,
        out_shape=jax.ShapeDtypeStruct((B, C, H, W), jnp.bfloat16),
        grid_spec=pltpu.PrefetchScalarGridSpec(
            num_scalar_prefetch=0,
            grid=(B // tb,),
            in_specs=[pl.BlockSpec((tb, C, H, W), lambda b: (b, 0, 0, 0))],
            out_specs=pl.BlockSpec((tb, C, H, W), lambda b: (b, 0, 0, 0)),
        ),
        compiler_params=pltpu.CompilerParams(
            dimension_semantics=("parallel",),
            vmem_limit_bytes=100 * 1024 * 1024,
        ),
    )(x)


def _pe_kernel(a_ref, w_ref, pos_ref, cls_ref, b_ref, out_ref):
    TB, Mp, K = a_ref.shape
    M = out_ref.shape[1]
    D = w_ref.shape[1]
    emb = jnp.dot(
        a_ref[...].reshape(TB * Mp, K), w_ref[...],
        preferred_element_type=jnp.float32,
    ).reshape(TB, Mp, D)
    pb = jnp.concatenate(
        [cls_ref[...] + pos_ref[0:1, :], pos_ref[1:, :] + b_ref[...]], axis=0)
    out_ref[...] = emb[:, :M, :] + pb[None]


def _vit_patch_embed(x, conv_w, conv_b, cls_token, pos_embed, patch_size,
                     *, batch_tile=8):
    B, C, H, W = x.shape
    ph, pw = patch_size
    gh, gw = H // ph, W // pw
    N = gh * gw
    D = conv_w.shape[0]
    K = C * ph * pw
    assert pos_embed.shape[1] == N + 1
    Mp = ((N + 1 + 7) // 8) * 8

    xc = _cast_bf16(x, batch_tile)
    patches = xc.reshape(B, C, gh, ph, gw, pw).transpose(0, 2, 4, 1, 3, 5)
    patches = patches.reshape(B, N, K)
    patches = jnp.pad(patches, ((0, 0), (1, Mp - 1 - N), (0, 0)))

    w_mat = conv_w.reshape(D, K).T.astype(jnp.bfloat16)

    TB = batch_tile
    grid = (B // TB,)

    out = pl.pallas_call(
        _pe_kernel,
        out_shape=jax.ShapeDtypeStruct((B, N + 1, D), x.dtype),
        grid_spec=pltpu.PrefetchScalarGridSpec(
            num_scalar_prefetch=0,
            grid=grid,
            in_specs=[
                pl.BlockSpec((TB, Mp, K), lambda b: (b, 0, 0)),
                pl.BlockSpec((K, D), lambda b: (0, 0)),
                pl.BlockSpec((N + 1, D), lambda b: (0, 0)),
                pl.BlockSpec((1, D), lambda b: (0, 0)),
                pl.BlockSpec((1, D), lambda b: (0, 0)),
            ],
            out_specs=pl.BlockSpec((TB, N + 1, D), lambda b: (b, 0, 0)),
        ),
        compiler_params=pltpu.CompilerParams(
            dimension_semantics=("parallel",),
            vmem_limit_bytes=100 * 1024 * 1024,
        ),
    )(patches, w_mat, pos_embed[0], cls_token.reshape(1, D),
      conv_b.reshape(1, D))
    return out


def kernel(x, conv_w, conv_b, cls_token, pos_embed):
    return _vit_patch_embed(x, conv_w, conv_b, cls_token, pos_embed, (16, 16))


# speedup vs baseline: 1.1635x; 1.0895x over previous
import jax
import jax.numpy as jnp
from jax.experimental import pallas as pl
from jax.experimental.pallas import tpu as pltpu


def _pe_kernel(a_ref, w_ref, pos_ref, cls_ref, b_ref, out_ref):
    TB, Mp, K = a_ref.shape
    M = out_ref.shape[1]
    D = w_ref.shape[1]
    emb = jnp.dot(
        a_ref[...].reshape(TB * Mp, K), w_ref[...],
        preferred_element_type=jnp.float32,
    ).reshape(TB, Mp, D)
    pb = jnp.concatenate(
        [cls_ref[...] + pos_ref[0:1, :], pos_ref[1:, :] + b_ref[...]], axis=0)
    out_ref[...] = emb[:, :M, :] + pb[None]


def _vit_patch_embed(x, conv_w, conv_b, cls_token, pos_embed, patch_size,
                     *, batch_tile=16):
    B, C, H, W = x.shape
    ph, pw = patch_size
    gh, gw = H // ph, W // pw
    N = gh * gw
    D = conv_w.shape[0]
    K = C * ph * pw
    assert pos_embed.shape[1] == N + 1
    Mp = ((N + 1 + 7) // 8) * 8

    xc = x.astype(jnp.bfloat16)
    patches = xc.reshape(B, C, gh, ph, gw, pw).transpose(0, 2, 4, 1, 3, 5)
    patches = patches.reshape(B, N, K)
    patches = jnp.pad(patches, ((0, 0), (1, Mp - 1 - N), (0, 0)))

    w_mat = conv_w.reshape(D, K).T.astype(jnp.bfloat16)

    TB = batch_tile
    grid = (B // TB,)

    out = pl.pallas_call(
        _pe_kernel,
        out_shape=jax.ShapeDtypeStruct((B, N + 1, D), x.dtype),
        grid_spec=pltpu.PrefetchScalarGridSpec(
            num_scalar_prefetch=0,
            grid=grid,
            in_specs=[
                pl.BlockSpec((TB, Mp, K), lambda b: (b, 0, 0)),
                pl.BlockSpec((K, D), lambda b: (0, 0)),
                pl.BlockSpec((N + 1, D), lambda b: (0, 0)),
                pl.BlockSpec((1, D), lambda b: (0, 0)),
                pl.BlockSpec((1, D), lambda b: (0, 0)),
            ],
            out_specs=pl.BlockSpec((TB, N + 1, D), lambda b: (b, 0, 0)),
        ),
        compiler_params=pltpu.CompilerParams(
            dimension_semantics=("parallel",),
            vmem_limit_bytes=100 * 1024 * 1024,
        ),
    )(patches, w_mat, pos_embed[0], cls_token.reshape(1, D),
      conv_b.reshape(1, D))
    return out


def kernel(x, conv_w, conv_b, cls_token, pos_embed):
    return _vit_patch_embed(x, conv_w, conv_b, cls_token, pos_embed, (16, 16))
